# Initial kernel scaffold; baseline (speedup 1.0000x reference)
import jax
import jax.numpy as jnp
from jax.experimental import pallas as pl


def kernel(x):
    raise NotImplementedError("write your pallas kernel here")


# speedup vs baseline: 1.0670x; 1.0670x over previous
import jax
import jax.numpy as jnp
from jax.experimental import pallas as pl

_PATCH = 16
_MASK_PROB = 0.5


def _mask_body(m_ref, x_ref, o_ref):
    for i in range(32):
        m = m_ref[i : i + 1, :]
        xs = x_ref[i * _PATCH : (i + 1) * _PATCH, :]
        o_ref[i * _PATCH : (i + 1) * _PATCH, :] = jnp.where(m != 0.0, 0.0, xs)


def kernel(x):
    img = x[0]
    H, W = img.shape
    nH, nW = H // _PATCH, W // _PATCH
    mkey = jax.random.key(12345)
    patch_mask = jax.random.uniform(mkey, (nH, nW)) < _MASK_PROB
    mask_cols = jnp.repeat(patch_mask, _PATCH, axis=1).astype(jnp.float32)

    out = pl.pallas_call(
        _mask_body,
        out_shape=jax.ShapeDtypeStruct((H, W), img.dtype),
    )(mask_cols, img)
    return out[None]
